# Initial kernel scaffold; baseline (speedup 1.0000x reference)
import jax
import jax.numpy as jnp
from jax.experimental import pallas as pl


def kernel(inputs, embedding):
    raise NotImplementedError("write your pallas kernel here")


# speedup vs baseline: 10.5713x; 10.5713x over previous
import jax
import jax.numpy as jnp
from jax.experimental import pallas as pl
from jax.experimental.pallas import tpu as pltpu
from jax.experimental.pallas import tpu_sc as plsc

_B = 16384
_D = 256
_N = 8192
_TB = 512
_NT = _B // _TB
_W = 128
_NW = 4096


def _dist_argmin_body(x_ref, et_ref, sx_ref, se_ref, idx_ref, loss_ref):
    step = pl.program_id(0)

    @pl.when(step == 0)
    def _():
        loss_ref[0, 0] = 0.0

    x = x_ref[...]
    mm = jax.lax.dot_general(
        x.astype(jnp.bfloat16), et_ref[...].astype(jnp.bfloat16),
        (((1,), (0,)), ((), ())),
        preferred_element_type=jnp.float32)
    sx = sx_ref[:, 0:1]
    se = se_ref[0:1, :]
    d = (sx + se) - 2.0 * mm
    inf = jnp.float32(jnp.inf)
    v = jnp.full((_TB, 1), inf, jnp.float32)
    vex = jnp.zeros((_TB, 1), jnp.float32)
    iacc = jnp.zeros((_TB, 1), jnp.int32)
    for lo in (0, _NW):
        dw = d[:, lo:lo + _NW]
        colw = lo + jax.lax.broadcasted_iota(jnp.int32, dw.shape, 1)
        dmin = jnp.min(dw, axis=1, keepdims=True)
        iw = jnp.min(jnp.where(dw == dmin, colw, _N), axis=1, keepdims=True)
        take = dmin < v
        v = jnp.where(take, dmin.astype(jnp.bfloat16).astype(jnp.float32), v)
        vex = jnp.where(take, dmin, vex)
        iacc = jnp.where(take, iw, iacc)
    idx_ref[0, 0, :] = iacc[:, 0]
    loss_ref[0, 0] += jnp.sum(vex)


def _dist_argmin(inputs, et, sxb, se8, interpret=False):
    return pl.pallas_call(
        _dist_argmin_body,
        grid=(_NT,),
        in_specs=[
            pl.BlockSpec((_TB, _D), lambda i: (i, 0)),
            pl.BlockSpec((_D, _N), lambda i: (0, 0)),
            pl.BlockSpec((_TB, 128), lambda i: (i, 0)),
            pl.BlockSpec((8, _N), lambda i: (0, 0)),
        ],
        out_specs=[
            pl.BlockSpec((1, 1, _TB), lambda i: (i, 0, 0)),
            pl.BlockSpec(memory_space=pltpu.SMEM, block_shape=(1, 1),
                         index_map=lambda i: (0, 0)),
        ],
        out_shape=[
            jax.ShapeDtypeStruct((_NT, 1, _TB), jnp.int32),
            jax.ShapeDtypeStruct((1, 1), jnp.float32),
        ],
        interpret=interpret,
    )(inputs, et, sxb, se8)


def _sc_gather(embedding, idx2):
    mesh = plsc.VectorSubcoreMesh(core_axis_name="core",
                                  subcore_axis_name="subcore")

    @pl.kernel(out_type=jax.ShapeDtypeStruct((_B, _D), embedding.dtype),
               mesh=mesh)
    def k(e_hbm, i_hbm, o_hbm):
        def body(i_vmem, o_vmem):
            pltpu.sync_copy(e_hbm.at[i_vmem.at[0]], o_vmem)

        pltpu.emit_pipeline(
            body,
            grid=(_B // _W,),
            in_specs=[pl.BlockSpec((1, _W), index_map=lambda i: (0, i))],
            out_specs=[pl.BlockSpec((_W, _D), index_map=lambda i: (i, 0))],
            core_axis_name=("core", "subcore"),
            dimension_semantics=(pltpu.PARALLEL,),
        )(i_hbm, o_hbm)

    return k(embedding, idx2)


def kernel(inputs, embedding):
    et = embedding.T
    sx = jnp.sum(inputs**2, axis=1)
    se = jnp.sum(embedding**2, axis=1)
    sxb = jnp.broadcast_to(sx[:, None], (_B, 128))
    se8 = jnp.broadcast_to(se[None, :], (8, _N))
    idx3, loss_sum = _dist_argmin(inputs, et, sxb, se8)
    e_r = embedding.astype(jnp.bfloat16).astype(jnp.float32)
    quantized = _sc_gather(e_r, idx3.reshape(1, _B))
    vq_loss = loss_sum[0, 0] * (1.25 / (_B * _D))
    return quantized, vq_loss
